# baseline (speedup 1.0000x reference)
import functools

import jax
import jax.numpy as jnp
from jax import lax
from jax.experimental import pallas as pl
from jax.experimental.pallas import tpu as pltpu
from jax.experimental.pallas import tpu_sc as plsc

N = 10000
E = 320000
D_IN = 128
D_HID = 128
N_CLS = 64

NC = 2
NS = 16
NW = NC * NS
RW = 320
NPAD = NW * RW
G = 128
L = 16


def _mm_body(x_ref, w_ref, o_ref):
    o_ref[...] = jnp.dot(x_ref[...], w_ref[...],
                         preferred_element_type=jnp.float32)


def _matmul(x, w):
    m, k = x.shape
    _, n = w.shape
    blk = 1024
    return pl.pallas_call(
        _mm_body,
        grid=(m // blk,),
        in_specs=[
            pl.BlockSpec((blk, k), lambda i: (i, 0)),
            pl.BlockSpec((k, n), lambda i: (0, 0)),
        ],
        out_specs=pl.BlockSpec((blk, n), lambda i: (i, 0)),
        out_shape=jax.ShapeDtypeStruct((m, n), jnp.float32),
    )(x, w)


def _mid_body(s_ref, b_ref, w_ref, o_ref):
    h = jnp.maximum(s_ref[...] + b_ref[...], 0.0)
    o_ref[...] = jnp.dot(h, w_ref[...], preferred_element_type=jnp.float32)


def _relu_linear(s, b1, w):
    m, k = s.shape
    _, n = w.shape
    blk = 1024
    return pl.pallas_call(
        _mid_body,
        grid=(m // blk,),
        in_specs=[
            pl.BlockSpec((blk, k), lambda i: (i, 0)),
            pl.BlockSpec((1, k), lambda i: (0, 0)),
            pl.BlockSpec((k, n), lambda i: (0, 0)),
        ],
        out_specs=pl.BlockSpec((blk, n), lambda i: (i, 0)),
        out_shape=jax.ShapeDtypeStruct((m, n), jnp.float32),
    )(s, b1[None, :], w)


@functools.partial(jax.jit, static_argnums=(6,))
def _spmm_sc(a, cols, rows, vals, starts, bias, d):
    nchunk = d // L
    mesh = plsc.VectorSubcoreMesh(core_axis_name="c", subcore_axis_name="s")

    @functools.partial(
        pl.kernel,
        out_type=jax.ShapeDtypeStruct((NPAD, d), jnp.float32),
        mesh=mesh,
        compiler_params=pltpu.CompilerParams(use_tc_tiling_on_sc=False,
                                             needs_layout_passes=False),
        scratch_types=[
            pltpu.VMEM((RW, d), jnp.float32),
            pltpu.VMEM((2, G), jnp.int32),
            pltpu.VMEM((2, G), jnp.int32),
            pltpu.VMEM((2, G), jnp.float32),
            pltpu.VMEM((2, G, d), jnp.float32),
            pltpu.VMEM((NW, L), jnp.int32),
            pltpu.VMEM((d,), jnp.float32),
            pltpu.SemaphoreType.DMA,
            pltpu.SemaphoreType.DMA,
            pltpu.SemaphoreType.DMA,
            pltpu.SemaphoreType.DMA,
        ],
    )
    def spmm(a_hbm, cols_hbm, rows_hbm, vals_hbm, starts_hbm, bias_hbm,
             o_hbm, acc, colb, rowb, valb, gbuf, startsb, biasb,
             sem_i0, sem_i1, sem_g0, sem_g1):
        sem_i = (sem_i0, sem_i1)
        sem_g = (sem_g0, sem_g1)
        wid = lax.axis_index("s") * NC + lax.axis_index("c")
        base = wid * RW
        pltpu.sync_copy(starts_hbm, startsb)
        pltpu.sync_copy(bias_hbm, biasb)
        se = startsb[wid, pl.ds(0, L)]
        es = se[0]
        ee = se[1]
        w0 = (es // G) * G
        nwin = (ee - w0 + (G - 1)) // G

        def start_idx(w, p):
            e0 = w0 + w * G
            pltpu.async_copy(cols_hbm.at[pl.ds(e0, G)], colb.at[p], sem_i[p])
            pltpu.async_copy(rows_hbm.at[pl.ds(e0, G)], rowb.at[p], sem_i[p])
            pltpu.async_copy(vals_hbm.at[pl.ds(e0, G)], valb.at[p], sem_i[p])

        def wait_idx(p):
            pltpu.make_async_copy(cols_hbm.at[pl.ds(0, G)], colb.at[p],
                                  sem_i[p]).wait()
            pltpu.make_async_copy(rows_hbm.at[pl.ds(0, G)], rowb.at[p],
                                  sem_i[p]).wait()
            pltpu.make_async_copy(vals_hbm.at[pl.ds(0, G)], valb.at[p],
                                  sem_i[p]).wait()

        def start_gather(p):
            pltpu.async_copy(a_hbm.at[colb.at[p]], gbuf.at[p], sem_g[p])

        def wait_gather(p):
            pltpu.make_async_copy(a_hbm.at[colb.at[p]], gbuf.at[p],
                                  sem_g[p]).wait()

        @pl.when(nwin > 0)
        def _():
            start_idx(0, 0)
            wait_idx(0)
            start_gather(0)

        @pl.when(nwin > 1)
        def _():
            start_idx(1, 1)

        @pl.loop(0, RW)
        def _(r):
            for k in range(nchunk):
                acc[r, pl.ds(L * k, L)] = biasb[pl.ds(L * k, L)]

        def _splat(vec, j):
            jsplat = jnp.full((L,), j, dtype=jnp.int32)
            return lax.gather(
                vec, jsplat[:, None],
                dimension_numbers=lax.GatherDimensionNumbers(
                    offset_dims=(), collapsed_slice_dims=(0,),
                    start_index_map=(0,)),
                slice_sizes=(1,),
                mode=lax.GatherScatterMode.PROMISE_IN_BOUNDS)

        def _flush(row, regs):
            for k in range(nchunk):
                acc[row, pl.ds(L * k, L)] = (acc[row, pl.ds(L * k, L)]
                                             + regs[k])

        def compute(p):
            def group_body(g, c):
                gb = g * L
                r_vec = rowb[p, pl.ds(gb, L)]
                lr_vec = jnp.clip(r_vec - base, 0, RW - 1)
                v_vec = jnp.where(
                    jnp.logical_and(r_vec >= base, r_vec < base + RW),
                    valb[p, pl.ds(gb, L)], 0.0)
                vjs = [_splat(v_vec, j) for j in range(L)]

                def span_fast(jlo, jhi, row):
                    regs = None
                    for j in range(jlo, jhi):
                        gj = [gbuf[p, gb + j, pl.ds(L * k, L)] * vjs[j]
                              for k in range(nchunk)]
                        regs = gj if regs is None else [
                            regs[k] + gj[k] for k in range(nchunk)]
                    _flush(row, regs)

                def span_slow(jlo, jhi):
                    cur = lr_vec[jlo]
                    regs = [gbuf[p, gb + jlo, pl.ds(L * k, L)] * vjs[jlo]
                            for k in range(nchunk)]
                    for j in range(jlo + 1, jhi):
                        lrj = lr_vec[j]
                        same = lrj == cur
                        prev_regs = regs
                        prev_cur = cur

                        @pl.when(jnp.logical_not(same))
                        def _():
                            _flush(prev_cur, prev_regs)

                        regs = [jnp.where(same, prev_regs[k], 0.0)
                                + gbuf[p, gb + j, pl.ds(L * k, L)] * vjs[j]
                                for k in range(nchunk)]
                        cur = lrj
                    _flush(cur, regs)

                first = lr_vec[0]
                last = lr_vec[L - 1]

                @pl.when(first == last)
                def _():
                    span_fast(0, L, first)

                @pl.when(first != last)
                def _():
                    for h in range(2):
                        hf = lr_vec[h * (L // 2)]
                        hl = lr_vec[h * (L // 2) + L // 2 - 1]

                        @pl.when(hf == hl)
                        def _(hf=hf, h=h):
                            span_fast(h * (L // 2), (h + 1) * (L // 2), hf)

                        @pl.when(hf != hl)
                        def _(h=h):
                            span_slow(h * (L // 2), (h + 1) * (L // 2))

                return c

            lax.fori_loop(0, G // L, group_body, 0)

        def outer(i, carry):
            for p in (0, 1):
                w = 2 * i + p

                @pl.when(w < nwin)
                def _():
                    @pl.when(w + 1 < nwin)
                    def _():
                        wait_idx(1 - p)
                        start_gather(1 - p)

                    wait_gather(p)
                    compute(p)

                    @pl.when(w + 2 < nwin)
                    def _():
                        start_idx(w + 2, p)
            return carry

        lax.fori_loop(0, (nwin + 1) // 2, outer, 0)
        pltpu.sync_copy(acc, o_hbm.at[pl.ds(base, RW)])

    return spmm(a, cols, rows, vals, starts, bias)


def kernel(x, propagation_adj, filter_vals, W1, b1, W2, b2,
           filter_rows, filter_cols):
    del propagation_adj
    xpad = jnp.pad(x, ((0, NPAD - N), (0, 0)))
    boundaries = jnp.arange(1, NW, dtype=jnp.int32) * RW
    starts_mid = jnp.searchsorted(filter_rows, boundaries).astype(jnp.int32)
    edges = jnp.concatenate([
        jnp.zeros((1,), jnp.int32),
        starts_mid,
        jnp.full((1,), E, jnp.int32),
    ])
    starts = jnp.pad(jnp.stack([edges[:-1], edges[1:]], axis=1),
                     ((0, 0), (0, L - 2)))
    zero_bias = jnp.zeros((D_HID,), jnp.float32)

    a = _matmul(xpad, W1.T)
    s1 = _spmm_sc(a, filter_cols, filter_rows, filter_vals, starts,
                  zero_bias, D_HID)
    y = _relu_linear(s1, b1, W2.T)
    out = _spmm_sc(y, filter_cols, filter_rows, filter_vals, starts,
                   b2, N_CLS)
    return out[:N]


# speedup vs baseline: 1.0170x; 1.0170x over previous
import functools

import jax
import jax.numpy as jnp
from jax import lax
from jax.experimental import pallas as pl
from jax.experimental.pallas import tpu as pltpu
from jax.experimental.pallas import tpu_sc as plsc

N = 10000
E = 320000
D_IN = 128
D_HID = 128
N_CLS = 64

NC = 2
NS = 16
NW = NC * NS
RW = 320
NPAD = NW * RW
G = 128
L = 16


def _mm_body(x_ref, w_ref, o_ref):
    o_ref[...] = jax.lax.dot_general(
        x_ref[...], w_ref[...], (((1,), (1,)), ((), ())),
        preferred_element_type=jnp.float32)


def _matmul(x, w, blk):
    m, k = x.shape
    n = w.shape[0]
    return pl.pallas_call(
        _mm_body,
        grid=(m // blk,),
        in_specs=[
            pl.BlockSpec((blk, k), lambda i: (i, 0)),
            pl.BlockSpec((n, k), lambda i: (0, 0)),
        ],
        out_specs=pl.BlockSpec((blk, n), lambda i: (i, 0)),
        out_shape=jax.ShapeDtypeStruct((m, n), jnp.float32),
    )(x, w)


def _mid_body(s_ref, b_ref, w_ref, o_ref):
    h = jnp.maximum(s_ref[...] + b_ref[...], 0.0)
    o_ref[...] = jax.lax.dot_general(
        h, w_ref[...], (((1,), (1,)), ((), ())),
        preferred_element_type=jnp.float32)


def _relu_linear(s, b1, w, blk):
    m, k = s.shape
    n = w.shape[0]
    return pl.pallas_call(
        _mid_body,
        grid=(m // blk,),
        in_specs=[
            pl.BlockSpec((blk, k), lambda i: (i, 0)),
            pl.BlockSpec((1, k), lambda i: (0, 0)),
            pl.BlockSpec((n, k), lambda i: (0, 0)),
        ],
        out_specs=pl.BlockSpec((blk, n), lambda i: (i, 0)),
        out_shape=jax.ShapeDtypeStruct((m, n), jnp.float32),
    )(s, b1[None, :], w)


@functools.partial(jax.jit, static_argnums=(6,))
def _spmm_sc(a, cols, rows, vals, starts, bias, d):
    nchunk = d // L
    mesh = plsc.VectorSubcoreMesh(core_axis_name="c", subcore_axis_name="s")

    @functools.partial(
        pl.kernel,
        out_type=jax.ShapeDtypeStruct((NPAD, d), jnp.float32),
        mesh=mesh,
        compiler_params=pltpu.CompilerParams(use_tc_tiling_on_sc=False,
                                             needs_layout_passes=False),
        scratch_types=[
            pltpu.VMEM((RW, d), jnp.float32),
            pltpu.VMEM((2, G), jnp.int32),
            pltpu.VMEM((2, G), jnp.int32),
            pltpu.VMEM((2, G), jnp.float32),
            pltpu.VMEM((2, G, d), jnp.float32),
            pltpu.VMEM((NW, L), jnp.int32),
            pltpu.VMEM((d,), jnp.float32),
            pltpu.SemaphoreType.DMA,
            pltpu.SemaphoreType.DMA,
            pltpu.SemaphoreType.DMA,
            pltpu.SemaphoreType.DMA,
        ],
    )
    def spmm(a_hbm, cols_hbm, rows_hbm, vals_hbm, starts_hbm, bias_hbm,
             o_hbm, acc, colb, rowb, valb, gbuf, startsb, biasb,
             sem_i0, sem_i1, sem_g0, sem_g1):
        sem_i = (sem_i0, sem_i1)
        sem_g = (sem_g0, sem_g1)
        wid = lax.axis_index("s") * NC + lax.axis_index("c")
        base = wid * RW
        pltpu.sync_copy(starts_hbm, startsb)
        pltpu.sync_copy(bias_hbm, biasb)
        se = startsb[wid, pl.ds(0, L)]
        es = se[0]
        ee = se[1]
        w0 = (es // G) * G
        nwin = (ee - w0 + (G - 1)) // G

        def start_idx(w, p):
            e0 = w0 + w * G
            pltpu.async_copy(cols_hbm.at[pl.ds(e0, G)], colb.at[p], sem_i[p])
            pltpu.async_copy(rows_hbm.at[pl.ds(e0, G)], rowb.at[p], sem_i[p])
            pltpu.async_copy(vals_hbm.at[pl.ds(e0, G)], valb.at[p], sem_i[p])

        def wait_idx(p):
            pltpu.make_async_copy(cols_hbm.at[pl.ds(0, G)], colb.at[p],
                                  sem_i[p]).wait()
            pltpu.make_async_copy(rows_hbm.at[pl.ds(0, G)], rowb.at[p],
                                  sem_i[p]).wait()
            pltpu.make_async_copy(vals_hbm.at[pl.ds(0, G)], valb.at[p],
                                  sem_i[p]).wait()

        def start_gather(p):
            pltpu.async_copy(a_hbm.at[colb.at[p]], gbuf.at[p], sem_g[p])

        def wait_gather(p):
            pltpu.make_async_copy(a_hbm.at[colb.at[p]], gbuf.at[p],
                                  sem_g[p]).wait()

        @pl.when(nwin > 0)
        def _():
            start_idx(0, 0)
            wait_idx(0)
            start_gather(0)

        @pl.when(nwin > 1)
        def _():
            start_idx(1, 1)

        @pl.loop(0, RW)
        def _(r):
            for k in range(nchunk):
                acc[r, pl.ds(L * k, L)] = biasb[pl.ds(L * k, L)]

        def _splat(vec, j):
            jsplat = jnp.full((L,), j, dtype=jnp.int32)
            return lax.gather(
                vec, jsplat[:, None],
                dimension_numbers=lax.GatherDimensionNumbers(
                    offset_dims=(), collapsed_slice_dims=(0,),
                    start_index_map=(0,)),
                slice_sizes=(1,),
                mode=lax.GatherScatterMode.PROMISE_IN_BOUNDS)

        def _flush(row, regs):
            for k in range(nchunk):
                acc[row, pl.ds(L * k, L)] = (acc[row, pl.ds(L * k, L)]
                                             + regs[k])

        def compute(p):
            def group_body(g, c):
                gb = g * L
                r_vec = rowb[p, pl.ds(gb, L)]
                lr_vec = jnp.clip(r_vec - base, 0, RW - 1)
                v_vec = jnp.where(
                    jnp.logical_and(r_vec >= base, r_vec < base + RW),
                    valb[p, pl.ds(gb, L)], 0.0)
                vjs = [_splat(v_vec, j) for j in range(L)]

                def span_fast(jlo, jhi, row):
                    regs = None
                    for j in range(jlo, jhi):
                        gj = [gbuf[p, gb + j, pl.ds(L * k, L)] * vjs[j]
                              for k in range(nchunk)]
                        regs = gj if regs is None else [
                            regs[k] + gj[k] for k in range(nchunk)]
                    _flush(row, regs)

                def span_slow(jlo, jhi):
                    cur = lr_vec[jlo]
                    regs = [gbuf[p, gb + jlo, pl.ds(L * k, L)] * vjs[jlo]
                            for k in range(nchunk)]
                    for j in range(jlo + 1, jhi):
                        lrj = lr_vec[j]
                        same = lrj == cur
                        prev_regs = regs
                        prev_cur = cur

                        @pl.when(jnp.logical_not(same))
                        def _():
                            _flush(prev_cur, prev_regs)

                        regs = [jnp.where(same, prev_regs[k], 0.0)
                                + gbuf[p, gb + j, pl.ds(L * k, L)] * vjs[j]
                                for k in range(nchunk)]
                        cur = lrj
                    _flush(cur, regs)

                first = lr_vec[0]
                last = lr_vec[L - 1]

                @pl.when(first == last)
                def _():
                    span_fast(0, L, first)

                @pl.when(first != last)
                def _():
                    for h in range(2):
                        hf = lr_vec[h * (L // 2)]
                        hl = lr_vec[h * (L // 2) + L // 2 - 1]

                        @pl.when(hf == hl)
                        def _(hf=hf, h=h):
                            span_fast(h * (L // 2), (h + 1) * (L // 2), hf)

                        @pl.when(hf != hl)
                        def _(h=h):
                            span_slow(h * (L // 2), (h + 1) * (L // 2))

                return c

            lax.fori_loop(0, G // L, group_body, 0)

        def outer(i, carry):
            for p in (0, 1):
                w = 2 * i + p

                @pl.when(w < nwin)
                def _():
                    wait_gather(p)

                    @pl.when(w + 1 < nwin)
                    def _():
                        wait_idx(1 - p)
                        start_gather(1 - p)

                    compute(p)

                    @pl.when(w + 2 < nwin)
                    def _():
                        start_idx(w + 2, p)
            return carry

        lax.fori_loop(0, (nwin + 1) // 2, outer, 0)
        pltpu.sync_copy(acc, o_hbm.at[pl.ds(base, RW)])

    return spmm(a, cols, rows, vals, starts, bias)


def kernel(x, propagation_adj, filter_vals, W1, b1, W2, b2,
           filter_rows, filter_cols):
    del propagation_adj
    boundaries = jnp.arange(1, NW, dtype=jnp.int32) * RW
    starts_mid = jnp.searchsorted(filter_rows, boundaries).astype(jnp.int32)
    edges = jnp.concatenate([
        jnp.zeros((1,), jnp.int32),
        starts_mid,
        jnp.full((1,), E, jnp.int32),
    ])
    starts = jnp.pad(jnp.stack([edges[:-1], edges[1:]], axis=1),
                     ((0, 0), (0, L - 2)))
    zero_bias = jnp.zeros((D_HID,), jnp.float32)

    a = _matmul(x, W1, 1000)
    s1 = _spmm_sc(a, filter_cols, filter_rows, filter_vals, starts,
                  zero_bias, D_HID)
    y = _relu_linear(s1, b1, W2, 1024)
    out = _spmm_sc(y, filter_cols, filter_rows, filter_vals, starts,
                   b2, N_CLS)
    return out[:N]


# speedup vs baseline: 1.0308x; 1.0136x over previous
import functools

import jax
import jax.numpy as jnp
from jax import lax
from jax.experimental import pallas as pl
from jax.experimental.pallas import tpu as pltpu
from jax.experimental.pallas import tpu_sc as plsc

N = 10000
E = 320000
D_IN = 128
D_HID = 128
N_CLS = 64

NC = 2
NS = 16
NW = NC * NS
RW = 320
NPAD = NW * RW
G = 128
L = 16


def _mm_body(x_ref, w_ref, o_ref):
    o_ref[...] = jnp.dot(x_ref[...], w_ref[...],
                         preferred_element_type=jnp.float32)


def _matmul(x, w):
    m, k = x.shape
    _, n = w.shape
    blk = 1024
    return pl.pallas_call(
        _mm_body,
        grid=(m // blk,),
        in_specs=[
            pl.BlockSpec((blk, k), lambda i: (i, 0)),
            pl.BlockSpec((k, n), lambda i: (0, 0)),
        ],
        out_specs=pl.BlockSpec((blk, n), lambda i: (i, 0)),
        out_shape=jax.ShapeDtypeStruct((m, n), jnp.float32),
    )(x, w)


def _mid_body(s_ref, b_ref, w_ref, o_ref):
    h = jnp.maximum(s_ref[...] + b_ref[...], 0.0)
    o_ref[...] = jnp.dot(h, w_ref[...], preferred_element_type=jnp.float32)


def _relu_linear(s, b1, w):
    m, k = s.shape
    _, n = w.shape
    blk = 1024
    return pl.pallas_call(
        _mid_body,
        grid=(m // blk,),
        in_specs=[
            pl.BlockSpec((blk, k), lambda i: (i, 0)),
            pl.BlockSpec((1, k), lambda i: (0, 0)),
            pl.BlockSpec((k, n), lambda i: (0, 0)),
        ],
        out_specs=pl.BlockSpec((blk, n), lambda i: (i, 0)),
        out_shape=jax.ShapeDtypeStruct((m, n), jnp.float32),
    )(s, b1[None, :], w)


@functools.partial(jax.jit, static_argnums=(6,))
def _spmm_sc(a, cols, rows, vals, starts, bias, d):
    nchunk = d // L
    mesh = plsc.VectorSubcoreMesh(core_axis_name="c", subcore_axis_name="s")

    @functools.partial(
        pl.kernel,
        out_type=jax.ShapeDtypeStruct((NPAD, d), jnp.float32),
        mesh=mesh,
        compiler_params=pltpu.CompilerParams(use_tc_tiling_on_sc=False,
                                             needs_layout_passes=False),
        scratch_types=[
            pltpu.VMEM((RW, d), jnp.float32),
            pltpu.VMEM((2, G), jnp.int32),
            pltpu.VMEM((2, G), jnp.int32),
            pltpu.VMEM((2, G), jnp.float32),
            pltpu.VMEM((2, G, d), jnp.float32),
            pltpu.VMEM((NW, L), jnp.int32),
            pltpu.VMEM((d,), jnp.float32),
            pltpu.SemaphoreType.DMA,
            pltpu.SemaphoreType.DMA,
            pltpu.SemaphoreType.DMA,
            pltpu.SemaphoreType.DMA,
        ],
    )
    def spmm(a_hbm, cols_hbm, rows_hbm, vals_hbm, starts_hbm, bias_hbm,
             o_hbm, acc, colb, rowb, valb, gbuf, startsb, biasb,
             sem_i0, sem_i1, sem_g0, sem_g1):
        sem_i = (sem_i0, sem_i1)
        sem_g = (sem_g0, sem_g1)
        wid = lax.axis_index("s") * NC + lax.axis_index("c")
        base = wid * RW
        pltpu.sync_copy(starts_hbm, startsb)
        pltpu.sync_copy(bias_hbm, biasb)
        se = startsb[wid, pl.ds(0, L)]
        es = se[0]
        ee = se[1]
        w0 = (es // G) * G
        nwin = (ee - w0 + (G - 1)) // G

        def start_idx(w, p):
            e0 = w0 + w * G
            pltpu.async_copy(cols_hbm.at[pl.ds(e0, G)], colb.at[p], sem_i[p])
            pltpu.async_copy(rows_hbm.at[pl.ds(e0, G)], rowb.at[p], sem_i[p])
            pltpu.async_copy(vals_hbm.at[pl.ds(e0, G)], valb.at[p], sem_i[p])

        def wait_idx(p):
            pltpu.make_async_copy(cols_hbm.at[pl.ds(0, G)], colb.at[p],
                                  sem_i[p]).wait()
            pltpu.make_async_copy(rows_hbm.at[pl.ds(0, G)], rowb.at[p],
                                  sem_i[p]).wait()
            pltpu.make_async_copy(vals_hbm.at[pl.ds(0, G)], valb.at[p],
                                  sem_i[p]).wait()

        def start_gather(p):
            pltpu.async_copy(a_hbm.at[colb.at[p]], gbuf.at[p], sem_g[p])

        def wait_gather(p):
            pltpu.make_async_copy(a_hbm.at[colb.at[p]], gbuf.at[p],
                                  sem_g[p]).wait()

        @pl.when(nwin > 0)
        def _():
            start_idx(0, 0)
            wait_idx(0)
            start_gather(0)

        @pl.when(nwin > 1)
        def _():
            start_idx(1, 1)

        @pl.loop(0, RW)
        def _(r):
            for k in range(nchunk):
                acc[r, pl.ds(L * k, L)] = biasb[pl.ds(L * k, L)]

        def _splat(vec, j):
            jsplat = jnp.full((L,), j, dtype=jnp.int32)
            return lax.gather(
                vec, jsplat[:, None],
                dimension_numbers=lax.GatherDimensionNumbers(
                    offset_dims=(), collapsed_slice_dims=(0,),
                    start_index_map=(0,)),
                slice_sizes=(1,),
                mode=lax.GatherScatterMode.PROMISE_IN_BOUNDS)

        def _flush(row, regs):
            for k in range(nchunk):
                acc[row, pl.ds(L * k, L)] = (acc[row, pl.ds(L * k, L)]
                                             + regs[k])

        def compute(p):
            def group_body(g, c):
                gb = g * L
                r_vec = rowb[p, pl.ds(gb, L)]
                lr_vec = jnp.clip(r_vec - base, 0, RW - 1)
                v_vec = jnp.where(
                    jnp.logical_and(r_vec >= base, r_vec < base + RW),
                    valb[p, pl.ds(gb, L)], 0.0)
                vjs = [_splat(v_vec, j) for j in range(L)]

                def span_fast(jlo, jhi, row):
                    regs = None
                    for j in range(jlo, jhi):
                        gj = [gbuf[p, gb + j, pl.ds(L * k, L)] * vjs[j]
                              for k in range(nchunk)]
                        regs = gj if regs is None else [
                            regs[k] + gj[k] for k in range(nchunk)]
                    _flush(row, regs)

                def span_slow(jlo, jhi):
                    cur = lr_vec[jlo]
                    regs = [gbuf[p, gb + jlo, pl.ds(L * k, L)] * vjs[jlo]
                            for k in range(nchunk)]
                    for j in range(jlo + 1, jhi):
                        lrj = lr_vec[j]
                        same = lrj == cur
                        prev_regs = regs
                        prev_cur = cur

                        @pl.when(jnp.logical_not(same))
                        def _():
                            _flush(prev_cur, prev_regs)

                        regs = [jnp.where(same, prev_regs[k], 0.0)
                                + gbuf[p, gb + j, pl.ds(L * k, L)] * vjs[j]
                                for k in range(nchunk)]
                        cur = lrj
                    _flush(cur, regs)

                first = lr_vec[0]
                last = lr_vec[L - 1]

                @pl.when(first == last)
                def _():
                    span_fast(0, L, first)

                @pl.when(first != last)
                def _():
                    for h in range(2):
                        hf = lr_vec[h * (L // 2)]
                        hl = lr_vec[h * (L // 2) + L // 2 - 1]

                        @pl.when(hf == hl)
                        def _(hf=hf, h=h):
                            span_fast(h * (L // 2), (h + 1) * (L // 2), hf)

                        @pl.when(hf != hl)
                        def _(h=h):
                            span_slow(h * (L // 2), (h + 1) * (L // 2))

                return c

            lax.fori_loop(0, G // L, group_body, 0)

        def outer(i, carry):
            for p in (0, 1):
                w = 2 * i + p

                @pl.when(w < nwin)
                def _():
                    wait_gather(p)

                    @pl.when(w + 1 < nwin)
                    def _():
                        wait_idx(1 - p)
                        start_gather(1 - p)

                    compute(p)

                    @pl.when(w + 2 < nwin)
                    def _():
                        start_idx(w + 2, p)
            return carry

        lax.fori_loop(0, (nwin + 1) // 2, outer, 0)
        pltpu.sync_copy(acc, o_hbm.at[pl.ds(base, RW)])

    return spmm(a, cols, rows, vals, starts, bias)


def kernel(x, propagation_adj, filter_vals, W1, b1, W2, b2,
           filter_rows, filter_cols):
    del propagation_adj
    xpad = jnp.pad(x, ((0, NPAD - N), (0, 0)))
    boundaries = jnp.arange(1, NW, dtype=jnp.int32) * RW
    starts_mid = jnp.searchsorted(filter_rows, boundaries).astype(jnp.int32)
    edges = jnp.concatenate([
        jnp.zeros((1,), jnp.int32),
        starts_mid,
        jnp.full((1,), E, jnp.int32),
    ])
    starts = jnp.pad(jnp.stack([edges[:-1], edges[1:]], axis=1),
                     ((0, 0), (0, L - 2)))
    zero_bias = jnp.zeros((D_HID,), jnp.float32)

    a = _matmul(xpad, W1.T)
    s1 = _spmm_sc(a, filter_cols, filter_rows, filter_vals, starts,
                  zero_bias, D_HID)
    y = _relu_linear(s1, b1, W2.T)
    out = _spmm_sc(y, filter_cols, filter_rows, filter_vals, starts,
                   b2, N_CLS)
    return out[:N]
